# baseline (speedup 1.0000x reference)
import functools

import jax
import jax.numpy as jnp
from jax import lax
from jax.experimental import pallas as pl
from jax.experimental.pallas import tpu as pltpu
from jax.experimental.pallas import tpu_sc as plsc

S1 = 4
S2 = 12
D = 64
NSEL = 4
NC, NS = 2, 16
NW = NC * NS
BLKB = 128
W2 = S2 * D
QS, QP, QO, QH, QT = 0, 1, 2, 4, 8


def _mesh():
    return plsc.VectorSubcoreMesh(
        core_axis_name="c", subcore_axis_name="s",
        num_cores=NC, num_subcores=NS)


def _build_sc_index(B):
    bpw = B // NW

    @functools.partial(
        pl.kernel,
        mesh=_mesh(),
        compiler_params=pltpu.CompilerParams(
            needs_layout_passes=False, use_tc_tiling_on_sc=False),
        out_type=[
            jax.ShapeDtypeStruct((B,), jnp.int32),
            jax.ShapeDtypeStruct((B,), jnp.int32),
            jax.ShapeDtypeStruct((B,), jnp.int32),
            jax.ShapeDtypeStruct((NSEL * B,), jnp.int32),
            jax.ShapeDtypeStruct((NSEL * B,), jnp.int32),
            jax.ShapeDtypeStruct((B, 16), jnp.int32),
        ],
        scratch_types=[
            pltpu.VMEM((16, B // NW), jnp.float32),
            pltpu.VMEM((16, B // NW), jnp.int32),
            pltpu.VMEM((16, B // NW), jnp.int32),
            pltpu.VMEM((3, B // NW), jnp.int32),
            pltpu.VMEM((B // NW,), jnp.int32),
            pltpu.VMEM((B // NW,), jnp.int32),
            pltpu.VMEM((B // NW,), jnp.int32),
            pltpu.VMEM((NSEL * (B // NW),), jnp.int32),
            pltpu.VMEM((NSEL * (B // NW),), jnp.int32),
            pltpu.VMEM((B // NW, 16), jnp.int32),
        ],
    )
    def sc_index(timeT_h, nhT_h, ntT_h, spoT_h,
                 si_h, pi_h, oi_h, hi_h, ti_h, q_h,
                 timeT_v, nhT_v, ntT_v, spoT_v,
                 sidx_v, pidx_v, oidx_v, hidx_v, tidx_v, q_v):
        wid = lax.axis_index("s") * NC + lax.axis_index("c")
        base = wid * bpw

        pltpu.sync_copy(timeT_h.at[:, pl.ds(base, bpw)], timeT_v)
        pltpu.sync_copy(nhT_h.at[:, pl.ds(base, bpw)], nhT_v)
        pltpu.sync_copy(ntT_h.at[:, pl.ds(base, bpw)], ntT_v)
        pltpu.sync_copy(spoT_h.at[:, pl.ds(base, bpw)], spoT_v)

        for g in range(bpw // 16):
            sl = pl.ds(g * 16, 16)
            lanes = jnp.arange(16, dtype=jnp.int32) + (g * 16)
            fv = (timeT_v[1, sl] + 2.0 * timeT_v[2, sl]
                  + 3.0 * timeT_v[3, sl])
            fi = fv.astype(jnp.int32)

            sv = spoT_v[0, sl]
            pv = spoT_v[1, sl]
            ov = spoT_v[2, sl]
            plsc.store_scatter(sidx_v, [lanes],
                               lax.shift_right_logical(sv, 3))
            plsc.store_scatter(pidx_v, [lanes],
                               lax.shift_right_logical(pv, 3))
            plsc.store_scatter(oidx_v, [lanes],
                               lax.shift_right_logical(ov, 3))
            for col, vals in ((QS, sv), (QP, pv), (QO, ov)):
                plsc.store_scatter(
                    q_v, [lanes, jnp.full((16,), col, jnp.int32)],
                    jnp.bitwise_and(vals, 7))
            for k in range(NSEL):
                rowsel = NSEL * fi + k
                hv = plsc.load_gather(nhT_v, [rowsel, lanes])
                tv = plsc.load_gather(ntT_v, [rowsel, lanes])
                dst = lanes + (k * bpw)
                plsc.store_scatter(hidx_v, [dst],
                                   lax.shift_right_logical(hv, 3))
                plsc.store_scatter(tidx_v, [dst],
                                   lax.shift_right_logical(tv, 3))
                plsc.store_scatter(
                    q_v, [lanes, jnp.full((16,), QH + k, jnp.int32)],
                    jnp.bitwise_and(hv, 7))
                plsc.store_scatter(
                    q_v, [lanes, jnp.full((16,), QT + k, jnp.int32)],
                    jnp.bitwise_and(tv, 7))

        pltpu.sync_copy(sidx_v, si_h.at[pl.ds(base, bpw)])
        pltpu.sync_copy(pidx_v, pi_h.at[pl.ds(base, bpw)])
        pltpu.sync_copy(oidx_v, oi_h.at[pl.ds(base, bpw)])
        for k in range(NSEL):
            pltpu.sync_copy(hidx_v.at[pl.ds(k * bpw, bpw)],
                            hi_h.at[pl.ds(k * B + base, bpw)])
            pltpu.sync_copy(tidx_v.at[pl.ds(k * bpw, bpw)],
                            ti_h.at[pl.ds(k * B + base, bpw)])
        pltpu.sync_copy(q_v, q_h.at[pl.ds(base, bpw)])

    return sc_index


def _build_sc_fetch(B):
    bpw = B // NW

    @functools.partial(
        pl.kernel,
        mesh=_mesh(),
        compiler_params=pltpu.CompilerParams(
            needs_layout_passes=False, use_tc_tiling_on_sc=True),
        out_type=[
            jax.ShapeDtypeStruct((B, 2, 128), jnp.int32),
            jax.ShapeDtypeStruct((B, 2, 128), jnp.int32),
            jax.ShapeDtypeStruct((B, 2, 128), jnp.int32),
            jax.ShapeDtypeStruct((NSEL * B, 2, 128), jnp.int32),
            jax.ShapeDtypeStruct((NSEL * B, 2, 128), jnp.int32),
        ],
        scratch_types=[
            pltpu.VMEM((B // NW,), jnp.int32),
            pltpu.VMEM((B // NW,), jnp.int32),
            pltpu.VMEM((B // NW,), jnp.int32),
            pltpu.VMEM((NSEL * (B // NW),), jnp.int32),
            pltpu.VMEM((NSEL * (B // NW),), jnp.int32),
            pltpu.VMEM((B // NW, 2, 128), jnp.int32),
            pltpu.VMEM((B // NW, 2, 128), jnp.int32),
            pltpu.VMEM((B // NW, 2, 128), jnp.int32),
            pltpu.VMEM((NSEL * (B // NW), 2, 128), jnp.int32),
            pltpu.VMEM((NSEL * (B // NW), 2, 128), jnp.int32),
            pltpu.SemaphoreType.DMA,
        ],
    )
    def sc_fetch(ent_h, rel_h, si_h, pi_h, oi_h, hi_h, ti_h,
                 sraw_h, praw_h, oraw_h, hraw_h, traw_h,
                 sidx_v, pidx_v, oidx_v, hidx_v, tidx_v,
                 srow_v, prow_v, orow_v, hrow_v, trow_v, sem):
        wid = lax.axis_index("s") * NC + lax.axis_index("c")
        base = wid * bpw

        pltpu.sync_copy(si_h.at[pl.ds(base, bpw)], sidx_v)
        pltpu.sync_copy(pi_h.at[pl.ds(base, bpw)], pidx_v)
        pltpu.sync_copy(oi_h.at[pl.ds(base, bpw)], oidx_v)
        for k in range(NSEL):
            pltpu.sync_copy(hi_h.at[pl.ds(k * B + base, bpw)],
                            hidx_v.at[pl.ds(k * bpw, bpw)])
            pltpu.sync_copy(ti_h.at[pl.ds(k * B + base, bpw)],
                            tidx_v.at[pl.ds(k * bpw, bpw)])

        cps = pltpu.async_copy(ent_h.at[sidx_v], srow_v, sem)
        cpp = pltpu.async_copy(rel_h.at[pidx_v], prow_v, sem)
        cpo = pltpu.async_copy(ent_h.at[oidx_v], orow_v, sem)
        cph = pltpu.async_copy(ent_h.at[hidx_v], hrow_v, sem)
        cpt = pltpu.async_copy(ent_h.at[tidx_v], trow_v, sem)
        cps.wait()
        cpp.wait()
        cpo.wait()
        cph.wait()
        cpt.wait()

        pltpu.sync_copy(srow_v, sraw_h.at[pl.ds(base, bpw)])
        pltpu.sync_copy(prow_v, praw_h.at[pl.ds(base, bpw)])
        pltpu.sync_copy(orow_v, oraw_h.at[pl.ds(base, bpw)])
        for k in range(NSEL):
            pltpu.sync_copy(hrow_v.at[pl.ds(k * bpw, bpw)],
                            hraw_h.at[pl.ds(k * B + base, bpw)])
            pltpu.sync_copy(trow_v.at[pl.ds(k * bpw, bpw)],
                            traw_h.at[pl.ds(k * B + base, bpw)])

    return sc_fetch


def _prep_body(l1e_ref, l2e_ref, l1r_ref, l2r_ref, cee_ref, cer_ref):
    for s in range(S1):
        cee_ref[s * W2:(s + 1) * W2, :] = jnp.dot(
            l2e_ref[...], l1e_ref[s], preferred_element_type=jnp.float32)
        cer_ref[s * W2:(s + 1) * W2, :] = jnp.dot(
            l2r_ref[...], l1r_ref[s], preferred_element_type=jnp.float32)


def _dot_t(a, b):
    return lax.dot_general(a, b, (((1,), (1,)), ((), ())),
                           preferred_element_type=jnp.float32)


def _select_rows(raw_ref, qcol):
    acc = jnp.where(qcol == 0, raw_ref[:, 0, 0:32], 0)
    for m in range(1, 8):
        sl = raw_ref[:, m >> 2, (m & 3) * 32:(m & 3) * 32 + 32]
        acc = jnp.bitwise_or(acc, jnp.where(qcol == m, sl, 0))
    lo = lax.bitcast_convert_type(lax.shift_left(acc, 16), jnp.float32)
    hi = lax.bitcast_convert_type(
        jnp.bitwise_and(acc, jnp.int32(-65536)), jnp.float32)
    return jnp.concatenate([lo, hi], axis=1)


def _main_body(timeT_ref, q_ref, sraw_ref, praw_ref, oraw_ref,
               ghraw_ref, gtraw_ref, cee_ref, cer_ref,
               s_out, p_out, o_out, h_out, t_out):
    cee = cee_ref[...]
    for k in range(NSEL):
        h_out[k] = _dot_t(cee, _select_rows(ghraw_ref.at[k],
                                            q_ref[:, QH + k:QH + k + 1]))
        t_out[k] = _dot_t(cee, _select_rows(gtraw_ref.at[k],
                                            q_ref[:, QT + k:QT + k + 1]))
    tm = timeT_ref[...]

    def timesel(full):
        acc = tm[0:1, :] * full[0:W2, :]
        for s in range(1, S1):
            acc = acc + tm[s:s + 1, :] * full[s * W2:(s + 1) * W2, :]
        return acc

    s_out[...] = timesel(_dot_t(cee, _select_rows(sraw_ref,
                                                  q_ref[:, QS:QS + 1])))
    o_out[...] = timesel(_dot_t(cee, _select_rows(oraw_ref,
                                                  q_ref[:, QO:QO + 1])))
    p_out[...] = timesel(_dot_t(cer_ref[...],
                                _select_rows(praw_ref,
                                             q_ref[:, QP:QP + 1])))


def kernel(spo, time, nh, nt, entity_embedding, relation_embedding,
           e_layer1, e_layer2, r_layer1, r_layer2):
    B = spo.shape[0]

    def pack(table):
        u = lax.bitcast_convert_type(table, jnp.uint32)
        r = (u + jnp.uint32(0x7FFF)
             + jnp.bitwise_and(u >> 16, jnp.uint32(1))) >> 16
        packed = jnp.bitwise_or(r[:, 0::2], r[:, 1::2] << 16)
        return lax.bitcast_convert_type(
            packed, jnp.int32).reshape(-1, 2, 128)

    ent3d = pack(entity_embedding)
    rel3d = pack(relation_embedding)
    timeT = time.astype(jnp.float32).T
    nhT = nh.astype(jnp.int32).T
    ntT = nt.astype(jnp.int32).T
    spoT = spo.astype(jnp.int32).T

    si, pi, oi, hi, ti, qmat = _build_sc_index(B)(timeT, nhT, ntT, spoT)
    sraw, praw, oraw, hraw, traw = _build_sc_fetch(B)(
        ent3d, rel3d, si, pi, oi, hi, ti)
    hraw = hraw.reshape(NSEL, B, 2, 128)
    traw = traw.reshape(NSEL, B, 2, 128)

    cee, cer = pl.pallas_call(
        _prep_body,
        out_shape=[
            jax.ShapeDtypeStruct((S1 * W2, D), jnp.float32),
            jax.ShapeDtypeStruct((S1 * W2, D), jnp.float32),
        ],
    )(e_layer1.reshape(S1, D, D), e_layer2,
      r_layer1.reshape(S1, D, D), r_layer2)
    perm = jnp.concatenate([jnp.arange(0, D, 2), jnp.arange(1, D, 2)])
    cee = cee[:, perm]
    cer = cer[:, perm]

    nblk = B // BLKB
    s_o, p_o, o_o, h_o, t_o = pl.pallas_call(
        _main_body,
        grid=(nblk,),
        in_specs=[
            pl.BlockSpec((16, BLKB), lambda i: (0, i)),
            pl.BlockSpec((BLKB, 16), lambda i: (i, 0)),
            pl.BlockSpec((BLKB, 2, 128), lambda i: (i, 0, 0)),
            pl.BlockSpec((BLKB, 2, 128), lambda i: (i, 0, 0)),
            pl.BlockSpec((BLKB, 2, 128), lambda i: (i, 0, 0)),
            pl.BlockSpec((NSEL, BLKB, 2, 128),
                         lambda i: (0, i, 0, 0)),
            pl.BlockSpec((NSEL, BLKB, 2, 128),
                         lambda i: (0, i, 0, 0)),
            pl.BlockSpec((S1 * W2, D), lambda i: (0, 0)),
            pl.BlockSpec((S1 * W2, D), lambda i: (0, 0)),
        ],
        out_specs=[
            pl.BlockSpec((W2, BLKB), lambda i: (0, i)),
            pl.BlockSpec((W2, BLKB), lambda i: (0, i)),
            pl.BlockSpec((W2, BLKB), lambda i: (0, i)),
            pl.BlockSpec((NSEL, S1 * W2, BLKB), lambda i: (0, 0, i)),
            pl.BlockSpec((NSEL, S1 * W2, BLKB), lambda i: (0, 0, i)),
        ],
        out_shape=[
            jax.ShapeDtypeStruct((W2, B), jnp.float32),
            jax.ShapeDtypeStruct((W2, B), jnp.float32),
            jax.ShapeDtypeStruct((W2, B), jnp.float32),
            jax.ShapeDtypeStruct((NSEL, S1 * W2, B), jnp.float32),
            jax.ShapeDtypeStruct((NSEL, S1 * W2, B), jnp.float32),
        ],
    )(timeT, qmat, sraw, praw, oraw, hraw, traw, cee, cer)

    def back(x, m):
        return jnp.transpose(x.reshape(-1, B), (1, 0)).reshape(B, S2, m, D)

    return (back(s_o, 1), back(p_o, 1), back(o_o, 1),
            back(h_o, 4 * NSEL), back(t_o, 4 * NSEL))


# speedup vs baseline: 2.4426x; 2.4426x over previous
import functools

import jax
import jax.numpy as jnp
from jax import lax
from jax.experimental import pallas as pl
from jax.experimental.pallas import tpu as pltpu
from jax.experimental.pallas import tpu_sc as plsc

S1 = 4
S2 = 12
D = 64
NSEL = 4
NC, NS = 2, 16
NW = NC * NS
BLKB = 128
W2 = S2 * D
QS, QP, QO, QH, QT = 0, 1, 2, 4, 8


def _mesh():
    return plsc.VectorSubcoreMesh(
        core_axis_name="c", subcore_axis_name="s",
        num_cores=NC, num_subcores=NS)


def _build_sc_index(B):
    bpw = B // NW

    @functools.partial(
        pl.kernel,
        mesh=_mesh(),
        compiler_params=pltpu.CompilerParams(
            needs_layout_passes=False, use_tc_tiling_on_sc=False),
        out_type=[
            jax.ShapeDtypeStruct((B,), jnp.int32),
            jax.ShapeDtypeStruct((B,), jnp.int32),
            jax.ShapeDtypeStruct((B,), jnp.int32),
            jax.ShapeDtypeStruct((NSEL * B,), jnp.int32),
            jax.ShapeDtypeStruct((NSEL * B,), jnp.int32),
            jax.ShapeDtypeStruct((B, 16), jnp.int32),
        ],
        scratch_types=[
            pltpu.VMEM((16, B // NW), jnp.float32),
            pltpu.VMEM((16, B // NW), jnp.int32),
            pltpu.VMEM((16, B // NW), jnp.int32),
            pltpu.VMEM((3, B // NW), jnp.int32),
            pltpu.VMEM((B // NW,), jnp.int32),
            pltpu.VMEM((B // NW,), jnp.int32),
            pltpu.VMEM((B // NW,), jnp.int32),
            pltpu.VMEM((NSEL * (B // NW),), jnp.int32),
            pltpu.VMEM((NSEL * (B // NW),), jnp.int32),
            pltpu.VMEM((B // NW, 16), jnp.int32),
        ],
    )
    def sc_index(timeT_h, nhT_h, ntT_h, spoT_h,
                 si_h, pi_h, oi_h, hi_h, ti_h, q_h,
                 timeT_v, nhT_v, ntT_v, spoT_v,
                 sidx_v, pidx_v, oidx_v, hidx_v, tidx_v, q_v):
        wid = lax.axis_index("s") * NC + lax.axis_index("c")
        base = wid * bpw

        pltpu.sync_copy(timeT_h.at[:, pl.ds(base, bpw)], timeT_v)
        pltpu.sync_copy(nhT_h.at[:, pl.ds(base, bpw)], nhT_v)
        pltpu.sync_copy(ntT_h.at[:, pl.ds(base, bpw)], ntT_v)
        pltpu.sync_copy(spoT_h.at[:, pl.ds(base, bpw)], spoT_v)

        for g in range(bpw // 16):
            sl = pl.ds(g * 16, 16)
            lanes = jnp.arange(16, dtype=jnp.int32) + (g * 16)
            fv = (timeT_v[1, sl] + 2.0 * timeT_v[2, sl]
                  + 3.0 * timeT_v[3, sl])
            fi = fv.astype(jnp.int32)

            sv = spoT_v[0, sl]
            pv = spoT_v[1, sl]
            ov = spoT_v[2, sl]
            plsc.store_scatter(sidx_v, [lanes],
                               lax.shift_right_logical(sv, 3))
            plsc.store_scatter(pidx_v, [lanes],
                               lax.shift_right_logical(pv, 3))
            plsc.store_scatter(oidx_v, [lanes],
                               lax.shift_right_logical(ov, 3))
            for col, vals in ((QS, sv), (QP, pv), (QO, ov)):
                plsc.store_scatter(
                    q_v, [lanes, jnp.full((16,), col, jnp.int32)],
                    jnp.bitwise_and(vals, 7))
            for k in range(NSEL):
                rowsel = NSEL * fi + k
                hv = plsc.load_gather(nhT_v, [rowsel, lanes])
                tv = plsc.load_gather(ntT_v, [rowsel, lanes])
                dst = lanes + (k * bpw)
                plsc.store_scatter(hidx_v, [dst],
                                   lax.shift_right_logical(hv, 3))
                plsc.store_scatter(tidx_v, [dst],
                                   lax.shift_right_logical(tv, 3))
                plsc.store_scatter(
                    q_v, [lanes, jnp.full((16,), QH + k, jnp.int32)],
                    jnp.bitwise_and(hv, 7))
                plsc.store_scatter(
                    q_v, [lanes, jnp.full((16,), QT + k, jnp.int32)],
                    jnp.bitwise_and(tv, 7))

        pltpu.sync_copy(sidx_v, si_h.at[pl.ds(base, bpw)])
        pltpu.sync_copy(pidx_v, pi_h.at[pl.ds(base, bpw)])
        pltpu.sync_copy(oidx_v, oi_h.at[pl.ds(base, bpw)])
        for k in range(NSEL):
            pltpu.sync_copy(hidx_v.at[pl.ds(k * bpw, bpw)],
                            hi_h.at[pl.ds(k * B + base, bpw)])
            pltpu.sync_copy(tidx_v.at[pl.ds(k * bpw, bpw)],
                            ti_h.at[pl.ds(k * B + base, bpw)])
        pltpu.sync_copy(q_v, q_h.at[pl.ds(base, bpw)])

    return sc_index


def _build_sc_fetch(B):
    bpw = B // NW

    @functools.partial(
        pl.kernel,
        mesh=_mesh(),
        compiler_params=pltpu.CompilerParams(
            needs_layout_passes=False, use_tc_tiling_on_sc=True),
        out_type=[
            jax.ShapeDtypeStruct((B, 2, 128), jnp.int32),
            jax.ShapeDtypeStruct((B, 2, 128), jnp.int32),
            jax.ShapeDtypeStruct((B, 2, 128), jnp.int32),
            jax.ShapeDtypeStruct((NSEL * B, 2, 128), jnp.int32),
            jax.ShapeDtypeStruct((NSEL * B, 2, 128), jnp.int32),
        ],
        scratch_types=[
            pltpu.VMEM((B // NW,), jnp.int32),
            pltpu.VMEM((B // NW,), jnp.int32),
            pltpu.VMEM((B // NW,), jnp.int32),
            pltpu.VMEM((NSEL * (B // NW),), jnp.int32),
            pltpu.VMEM((NSEL * (B // NW),), jnp.int32),
            pltpu.VMEM((B // NW, 2, 128), jnp.int32),
            pltpu.VMEM((B // NW, 2, 128), jnp.int32),
            pltpu.VMEM((B // NW, 2, 128), jnp.int32),
            pltpu.VMEM((NSEL * (B // NW), 2, 128), jnp.int32),
            pltpu.VMEM((NSEL * (B // NW), 2, 128), jnp.int32),
            pltpu.SemaphoreType.DMA,
        ],
    )
    def sc_fetch(ent_h, rel_h, si_h, pi_h, oi_h, hi_h, ti_h,
                 sraw_h, praw_h, oraw_h, hraw_h, traw_h,
                 sidx_v, pidx_v, oidx_v, hidx_v, tidx_v,
                 srow_v, prow_v, orow_v, hrow_v, trow_v, sem):
        wid = lax.axis_index("s") * NC + lax.axis_index("c")
        base = wid * bpw

        pltpu.sync_copy(si_h.at[pl.ds(base, bpw)], sidx_v)
        pltpu.sync_copy(pi_h.at[pl.ds(base, bpw)], pidx_v)
        pltpu.sync_copy(oi_h.at[pl.ds(base, bpw)], oidx_v)
        for k in range(NSEL):
            pltpu.sync_copy(hi_h.at[pl.ds(k * B + base, bpw)],
                            hidx_v.at[pl.ds(k * bpw, bpw)])
            pltpu.sync_copy(ti_h.at[pl.ds(k * B + base, bpw)],
                            tidx_v.at[pl.ds(k * bpw, bpw)])

        cps = pltpu.async_copy(ent_h.at[sidx_v], srow_v, sem)
        cpp = pltpu.async_copy(rel_h.at[pidx_v], prow_v, sem)
        cpo = pltpu.async_copy(ent_h.at[oidx_v], orow_v, sem)
        cph = pltpu.async_copy(ent_h.at[hidx_v], hrow_v, sem)
        cpt = pltpu.async_copy(ent_h.at[tidx_v], trow_v, sem)
        cps.wait()
        cpp.wait()
        cpo.wait()
        cph.wait()
        cpt.wait()

        pltpu.sync_copy(srow_v, sraw_h.at[pl.ds(base, bpw)])
        pltpu.sync_copy(prow_v, praw_h.at[pl.ds(base, bpw)])
        pltpu.sync_copy(orow_v, oraw_h.at[pl.ds(base, bpw)])
        for k in range(NSEL):
            pltpu.sync_copy(hrow_v.at[pl.ds(k * bpw, bpw)],
                            hraw_h.at[pl.ds(k * B + base, bpw)])
            pltpu.sync_copy(trow_v.at[pl.ds(k * bpw, bpw)],
                            traw_h.at[pl.ds(k * B + base, bpw)])

    return sc_fetch


def _prep_body(l1e_ref, l2e_ref, l1r_ref, l2r_ref, cee_ref, cer_ref):
    for s in range(S1):
        cee_ref[s * W2:(s + 1) * W2, :] = jnp.dot(
            l2e_ref[...], l1e_ref[s], preferred_element_type=jnp.float32)
        cer_ref[s * W2:(s + 1) * W2, :] = jnp.dot(
            l2r_ref[...], l1r_ref[s], preferred_element_type=jnp.float32)


def _dot_t(a, b):
    return lax.dot_general(a, b, (((1,), (1,)), ((), ())),
                           preferred_element_type=jnp.float32)


def _select_rows(raw_ref, qcol):
    acc = jnp.where(qcol == 0, raw_ref[:, 0, 0:32], 0)
    for m in range(1, 8):
        sl = raw_ref[:, m >> 2, (m & 3) * 32:(m & 3) * 32 + 32]
        acc = jnp.bitwise_or(acc, jnp.where(qcol == m, sl, 0))
    lo = lax.bitcast_convert_type(lax.shift_left(acc, 16), jnp.float32)
    hi = lax.bitcast_convert_type(
        jnp.bitwise_and(acc, jnp.int32(-65536)), jnp.float32)
    return jnp.concatenate([lo, hi], axis=1)


def _main_body(timeT_ref, q_ref, sraw_ref, praw_ref, oraw_ref,
               ghraw_ref, gtraw_ref, cee_ref, cer_ref,
               s_out, p_out, o_out, h_out, t_out):
    cee = cee_ref[...]
    for k in range(NSEL):
        h_out[k] = _dot_t(cee, _select_rows(ghraw_ref.at[k],
                                            q_ref[:, QH + k:QH + k + 1]))
        t_out[k] = _dot_t(cee, _select_rows(gtraw_ref.at[k],
                                            q_ref[:, QT + k:QT + k + 1]))
    tm = timeT_ref[...]

    def timesel(full):
        acc = tm[0:1, :] * full[0:W2, :]
        for s in range(1, S1):
            acc = acc + tm[s:s + 1, :] * full[s * W2:(s + 1) * W2, :]
        return acc

    s_out[...] = timesel(_dot_t(cee, _select_rows(sraw_ref,
                                                  q_ref[:, QS:QS + 1])))
    o_out[...] = timesel(_dot_t(cee, _select_rows(oraw_ref,
                                                  q_ref[:, QO:QO + 1])))
    p_out[...] = timesel(_dot_t(cer_ref[...],
                                _select_rows(praw_ref,
                                             q_ref[:, QP:QP + 1])))


def kernel(spo, time, nh, nt, entity_embedding, relation_embedding,
           e_layer1, e_layer2, r_layer1, r_layer2):
    B = spo.shape[0]

    def pack(table):
        u = lax.bitcast_convert_type(table, jnp.uint32)
        r = (u + jnp.uint32(0x7FFF)
             + jnp.bitwise_and(u >> 16, jnp.uint32(1))) >> 16
        packed = jnp.bitwise_or(r[:, :D // 2], r[:, D // 2:] << 16)
        return lax.bitcast_convert_type(
            packed, jnp.int32).reshape(-1, 2, 128)

    ent3d = pack(entity_embedding)
    rel3d = pack(relation_embedding)
    timeT = time.astype(jnp.float32).T
    nhT = nh.astype(jnp.int32).T
    ntT = nt.astype(jnp.int32).T
    spoT = spo.astype(jnp.int32).T

    si, pi, oi, hi, ti, qmat = _build_sc_index(B)(timeT, nhT, ntT, spoT)
    sraw, praw, oraw, hraw, traw = _build_sc_fetch(B)(
        ent3d, rel3d, si, pi, oi, hi, ti)
    hraw = hraw.reshape(NSEL, B, 2, 128)
    traw = traw.reshape(NSEL, B, 2, 128)

    cee, cer = pl.pallas_call(
        _prep_body,
        out_shape=[
            jax.ShapeDtypeStruct((S1 * W2, D), jnp.float32),
            jax.ShapeDtypeStruct((S1 * W2, D), jnp.float32),
        ],
    )(e_layer1.reshape(S1, D, D), e_layer2,
      r_layer1.reshape(S1, D, D), r_layer2)
    nblk = B // BLKB
    s_o, p_o, o_o, h_o, t_o = pl.pallas_call(
        _main_body,
        grid=(nblk,),
        in_specs=[
            pl.BlockSpec((16, BLKB), lambda i: (0, i)),
            pl.BlockSpec((BLKB, 16), lambda i: (i, 0)),
            pl.BlockSpec((BLKB, 2, 128), lambda i: (i, 0, 0)),
            pl.BlockSpec((BLKB, 2, 128), lambda i: (i, 0, 0)),
            pl.BlockSpec((BLKB, 2, 128), lambda i: (i, 0, 0)),
            pl.BlockSpec((NSEL, BLKB, 2, 128),
                         lambda i: (0, i, 0, 0)),
            pl.BlockSpec((NSEL, BLKB, 2, 128),
                         lambda i: (0, i, 0, 0)),
            pl.BlockSpec((S1 * W2, D), lambda i: (0, 0)),
            pl.BlockSpec((S1 * W2, D), lambda i: (0, 0)),
        ],
        out_specs=[
            pl.BlockSpec((W2, BLKB), lambda i: (0, i)),
            pl.BlockSpec((W2, BLKB), lambda i: (0, i)),
            pl.BlockSpec((W2, BLKB), lambda i: (0, i)),
            pl.BlockSpec((NSEL, S1 * W2, BLKB), lambda i: (0, 0, i)),
            pl.BlockSpec((NSEL, S1 * W2, BLKB), lambda i: (0, 0, i)),
        ],
        out_shape=[
            jax.ShapeDtypeStruct((W2, B), jnp.float32),
            jax.ShapeDtypeStruct((W2, B), jnp.float32),
            jax.ShapeDtypeStruct((W2, B), jnp.float32),
            jax.ShapeDtypeStruct((NSEL, S1 * W2, B), jnp.float32),
            jax.ShapeDtypeStruct((NSEL, S1 * W2, B), jnp.float32),
        ],
    )(timeT, qmat, sraw, praw, oraw, hraw, traw, cee, cer)

    def back(x, m):
        return jnp.transpose(x.reshape(-1, B), (1, 0)).reshape(B, S2, m, D)

    return (back(s_o, 1), back(p_o, 1), back(o_o, 1),
            back(h_o, 4 * NSEL), back(t_o, 4 * NSEL))


# speedup vs baseline: 2.5607x; 1.0483x over previous
import functools

import jax
import jax.numpy as jnp
from jax import lax
from jax.experimental import pallas as pl
from jax.experimental.pallas import tpu as pltpu
from jax.experimental.pallas import tpu_sc as plsc

S1 = 4
S2 = 12
D = 64
NSEL = 4
NC, NS = 2, 16
NW = NC * NS
BLKB = 128
W2 = S2 * D
QS, QP, QO, QH, QT = 0, 1, 2, 4, 8


def _mesh():
    return plsc.VectorSubcoreMesh(
        core_axis_name="c", subcore_axis_name="s",
        num_cores=NC, num_subcores=NS)


def _build_sc_index(B):
    bpw = B // NW

    @functools.partial(
        pl.kernel,
        mesh=_mesh(),
        compiler_params=pltpu.CompilerParams(
            needs_layout_passes=False, use_tc_tiling_on_sc=False),
        out_type=[
            jax.ShapeDtypeStruct((B,), jnp.int32),
            jax.ShapeDtypeStruct((B,), jnp.int32),
            jax.ShapeDtypeStruct((B,), jnp.int32),
            jax.ShapeDtypeStruct((NSEL * B,), jnp.int32),
            jax.ShapeDtypeStruct((NSEL * B,), jnp.int32),
            jax.ShapeDtypeStruct((B, 16), jnp.int32),
        ],
        scratch_types=[
            pltpu.VMEM((16, B // NW), jnp.float32),
            pltpu.VMEM((16, B // NW), jnp.int32),
            pltpu.VMEM((16, B // NW), jnp.int32),
            pltpu.VMEM((3, B // NW), jnp.int32),
            pltpu.VMEM((B // NW,), jnp.int32),
            pltpu.VMEM((B // NW,), jnp.int32),
            pltpu.VMEM((B // NW,), jnp.int32),
            pltpu.VMEM((NSEL * (B // NW),), jnp.int32),
            pltpu.VMEM((NSEL * (B // NW),), jnp.int32),
            pltpu.VMEM((B // NW, 16), jnp.int32),
        ],
    )
    def sc_index(timeT_h, nhT_h, ntT_h, spoT_h,
                 si_h, pi_h, oi_h, hi_h, ti_h, q_h,
                 timeT_v, nhT_v, ntT_v, spoT_v,
                 sidx_v, pidx_v, oidx_v, hidx_v, tidx_v, q_v):
        wid = lax.axis_index("s") * NC + lax.axis_index("c")
        base = wid * bpw

        pltpu.sync_copy(timeT_h.at[:, pl.ds(base, bpw)], timeT_v)
        pltpu.sync_copy(nhT_h.at[:, pl.ds(base, bpw)], nhT_v)
        pltpu.sync_copy(ntT_h.at[:, pl.ds(base, bpw)], ntT_v)
        pltpu.sync_copy(spoT_h.at[:, pl.ds(base, bpw)], spoT_v)

        for g in range(bpw // 16):
            sl = pl.ds(g * 16, 16)
            lanes = jnp.arange(16, dtype=jnp.int32) + (g * 16)
            fv = (timeT_v[1, sl] + 2.0 * timeT_v[2, sl]
                  + 3.0 * timeT_v[3, sl])
            fi = fv.astype(jnp.int32)

            sv = spoT_v[0, sl]
            pv = spoT_v[1, sl]
            ov = spoT_v[2, sl]
            plsc.store_scatter(sidx_v, [lanes],
                               lax.shift_right_logical(sv, 2))
            plsc.store_scatter(pidx_v, [lanes],
                               lax.shift_right_logical(pv, 2))
            plsc.store_scatter(oidx_v, [lanes],
                               lax.shift_right_logical(ov, 2))
            for col, vals in ((QS, sv), (QP, pv), (QO, ov)):
                plsc.store_scatter(
                    q_v, [lanes, jnp.full((16,), col, jnp.int32)],
                    jnp.bitwise_and(vals, 3))
            for k in range(NSEL):
                rowsel = NSEL * fi + k
                hv = plsc.load_gather(nhT_v, [rowsel, lanes])
                tv = plsc.load_gather(ntT_v, [rowsel, lanes])
                dst = lanes + (k * bpw)
                plsc.store_scatter(hidx_v, [dst],
                                   lax.shift_right_logical(hv, 2))
                plsc.store_scatter(tidx_v, [dst],
                                   lax.shift_right_logical(tv, 2))
                plsc.store_scatter(
                    q_v, [lanes, jnp.full((16,), QH + k, jnp.int32)],
                    jnp.bitwise_and(hv, 3))
                plsc.store_scatter(
                    q_v, [lanes, jnp.full((16,), QT + k, jnp.int32)],
                    jnp.bitwise_and(tv, 3))

        pltpu.sync_copy(sidx_v, si_h.at[pl.ds(base, bpw)])
        pltpu.sync_copy(pidx_v, pi_h.at[pl.ds(base, bpw)])
        pltpu.sync_copy(oidx_v, oi_h.at[pl.ds(base, bpw)])
        for k in range(NSEL):
            pltpu.sync_copy(hidx_v.at[pl.ds(k * bpw, bpw)],
                            hi_h.at[pl.ds(k * B + base, bpw)])
            pltpu.sync_copy(tidx_v.at[pl.ds(k * bpw, bpw)],
                            ti_h.at[pl.ds(k * B + base, bpw)])
        pltpu.sync_copy(q_v, q_h.at[pl.ds(base, bpw)])

    return sc_index


def _build_sc_fetch(B):
    bpw = B // NW

    @functools.partial(
        pl.kernel,
        mesh=_mesh(),
        compiler_params=pltpu.CompilerParams(
            needs_layout_passes=False, use_tc_tiling_on_sc=True),
        out_type=[
            jax.ShapeDtypeStruct((B, 128), jnp.int32),
            jax.ShapeDtypeStruct((B, 128), jnp.int32),
            jax.ShapeDtypeStruct((B, 128), jnp.int32),
            jax.ShapeDtypeStruct((NSEL * B, 128), jnp.int32),
            jax.ShapeDtypeStruct((NSEL * B, 128), jnp.int32),
        ],
        scratch_types=[
            pltpu.VMEM((B // NW,), jnp.int32),
            pltpu.VMEM((B // NW,), jnp.int32),
            pltpu.VMEM((B // NW,), jnp.int32),
            pltpu.VMEM((NSEL * (B // NW),), jnp.int32),
            pltpu.VMEM((NSEL * (B // NW),), jnp.int32),
            pltpu.VMEM((B // NW, 128), jnp.int32),
            pltpu.VMEM((B // NW, 128), jnp.int32),
            pltpu.VMEM((B // NW, 128), jnp.int32),
            pltpu.VMEM((NSEL * (B // NW), 128), jnp.int32),
            pltpu.VMEM((NSEL * (B // NW), 128), jnp.int32),
            pltpu.SemaphoreType.DMA,
        ],
    )
    def sc_fetch(ent_h, rel_h, si_h, pi_h, oi_h, hi_h, ti_h,
                 sraw_h, praw_h, oraw_h, hraw_h, traw_h,
                 sidx_v, pidx_v, oidx_v, hidx_v, tidx_v,
                 srow_v, prow_v, orow_v, hrow_v, trow_v, sem):
        wid = lax.axis_index("s") * NC + lax.axis_index("c")
        base = wid * bpw

        pltpu.sync_copy(si_h.at[pl.ds(base, bpw)], sidx_v)
        pltpu.sync_copy(pi_h.at[pl.ds(base, bpw)], pidx_v)
        pltpu.sync_copy(oi_h.at[pl.ds(base, bpw)], oidx_v)
        for k in range(NSEL):
            pltpu.sync_copy(hi_h.at[pl.ds(k * B + base, bpw)],
                            hidx_v.at[pl.ds(k * bpw, bpw)])
            pltpu.sync_copy(ti_h.at[pl.ds(k * B + base, bpw)],
                            tidx_v.at[pl.ds(k * bpw, bpw)])

        cps = pltpu.async_copy(ent_h.at[sidx_v], srow_v, sem)
        cpp = pltpu.async_copy(rel_h.at[pidx_v], prow_v, sem)
        cpo = pltpu.async_copy(ent_h.at[oidx_v], orow_v, sem)
        cph = pltpu.async_copy(ent_h.at[hidx_v], hrow_v, sem)
        cpt = pltpu.async_copy(ent_h.at[tidx_v], trow_v, sem)
        cps.wait()
        cpp.wait()
        cpo.wait()
        cph.wait()
        cpt.wait()

        pltpu.sync_copy(srow_v, sraw_h.at[pl.ds(base, bpw)])
        pltpu.sync_copy(prow_v, praw_h.at[pl.ds(base, bpw)])
        pltpu.sync_copy(orow_v, oraw_h.at[pl.ds(base, bpw)])
        for k in range(NSEL):
            pltpu.sync_copy(hrow_v.at[pl.ds(k * bpw, bpw)],
                            hraw_h.at[pl.ds(k * B + base, bpw)])
            pltpu.sync_copy(trow_v.at[pl.ds(k * bpw, bpw)],
                            traw_h.at[pl.ds(k * B + base, bpw)])

    return sc_fetch


def _prep_body(l1e_ref, l2e_ref, l1r_ref, l2r_ref, cee_ref, cer_ref):
    for s in range(S1):
        cee_ref[s * W2:(s + 1) * W2, :] = jnp.dot(
            l2e_ref[...], l1e_ref[s], preferred_element_type=jnp.float32)
        cer_ref[s * W2:(s + 1) * W2, :] = jnp.dot(
            l2r_ref[...], l1r_ref[s], preferred_element_type=jnp.float32)


def _dot_t(a, b):
    return lax.dot_general(a, b, (((1,), (1,)), ((), ())),
                           preferred_element_type=jnp.float32)


def _select_rows(raw_ref, qcol):
    acc = jnp.where(qcol == 0, raw_ref[:, 0:32], 0)
    for m in range(1, 4):
        sl = raw_ref[:, m * 32:(m + 1) * 32]
        acc = jnp.bitwise_or(acc, jnp.where(qcol == m, sl, 0))
    lo = lax.bitcast_convert_type(lax.shift_left(acc, 16), jnp.float32)
    hi = lax.bitcast_convert_type(
        jnp.bitwise_and(acc, jnp.int32(-65536)), jnp.float32)
    return jnp.concatenate([lo, hi], axis=1)


def _main_body(timeT_ref, q_ref, sraw_ref, praw_ref, oraw_ref,
               ghraw_ref, gtraw_ref, cee_ref, cer_ref,
               s_out, p_out, o_out, h_out, t_out):
    cee = cee_ref[...]
    for k in range(NSEL):
        h_out[k] = _dot_t(cee, _select_rows(ghraw_ref.at[k],
                                            q_ref[:, QH + k:QH + k + 1]))
        t_out[k] = _dot_t(cee, _select_rows(gtraw_ref.at[k],
                                            q_ref[:, QT + k:QT + k + 1]))
    tm = timeT_ref[...]

    def timesel(full):
        acc = tm[0:1, :] * full[0:W2, :]
        for s in range(1, S1):
            acc = acc + tm[s:s + 1, :] * full[s * W2:(s + 1) * W2, :]
        return acc

    s_out[...] = timesel(_dot_t(cee, _select_rows(sraw_ref,
                                                  q_ref[:, QS:QS + 1])))
    o_out[...] = timesel(_dot_t(cee, _select_rows(oraw_ref,
                                                  q_ref[:, QO:QO + 1])))
    p_out[...] = timesel(_dot_t(cer_ref[...],
                                _select_rows(praw_ref,
                                             q_ref[:, QP:QP + 1])))


def kernel(spo, time, nh, nt, entity_embedding, relation_embedding,
           e_layer1, e_layer2, r_layer1, r_layer2):
    B = spo.shape[0]

    def pack(table):
        u = lax.bitcast_convert_type(table, jnp.uint32)
        r = (u + jnp.uint32(0x7FFF)
             + jnp.bitwise_and(u >> 16, jnp.uint32(1))) >> 16
        packed = jnp.bitwise_or(r[:, :D // 2], r[:, D // 2:] << 16)
        return lax.bitcast_convert_type(packed, jnp.int32).reshape(-1, 128)

    ent3d = pack(entity_embedding)
    rel3d = pack(relation_embedding)
    timeT = time.astype(jnp.float32).T
    nhT = nh.astype(jnp.int32).T
    ntT = nt.astype(jnp.int32).T
    spoT = spo.astype(jnp.int32).T

    si, pi, oi, hi, ti, qmat = _build_sc_index(B)(timeT, nhT, ntT, spoT)
    sraw, praw, oraw, hraw, traw = _build_sc_fetch(B)(
        ent3d, rel3d, si, pi, oi, hi, ti)
    hraw = hraw.reshape(NSEL, B, 128)
    traw = traw.reshape(NSEL, B, 128)

    cee, cer = pl.pallas_call(
        _prep_body,
        out_shape=[
            jax.ShapeDtypeStruct((S1 * W2, D), jnp.float32),
            jax.ShapeDtypeStruct((S1 * W2, D), jnp.float32),
        ],
    )(e_layer1.reshape(S1, D, D), e_layer2,
      r_layer1.reshape(S1, D, D), r_layer2)
    nblk = B // BLKB
    s_o, p_o, o_o, h_o, t_o = pl.pallas_call(
        _main_body,
        grid=(nblk,),
        in_specs=[
            pl.BlockSpec((16, BLKB), lambda i: (0, i)),
            pl.BlockSpec((BLKB, 16), lambda i: (i, 0)),
            pl.BlockSpec((BLKB, 128), lambda i: (i, 0)),
            pl.BlockSpec((BLKB, 128), lambda i: (i, 0)),
            pl.BlockSpec((BLKB, 128), lambda i: (i, 0)),
            pl.BlockSpec((NSEL, BLKB, 128),
                         lambda i: (0, i, 0)),
            pl.BlockSpec((NSEL, BLKB, 128),
                         lambda i: (0, i, 0)),
            pl.BlockSpec((S1 * W2, D), lambda i: (0, 0)),
            pl.BlockSpec((S1 * W2, D), lambda i: (0, 0)),
        ],
        out_specs=[
            pl.BlockSpec((W2, BLKB), lambda i: (0, i)),
            pl.BlockSpec((W2, BLKB), lambda i: (0, i)),
            pl.BlockSpec((W2, BLKB), lambda i: (0, i)),
            pl.BlockSpec((NSEL, S1 * W2, BLKB), lambda i: (0, 0, i)),
            pl.BlockSpec((NSEL, S1 * W2, BLKB), lambda i: (0, 0, i)),
        ],
        out_shape=[
            jax.ShapeDtypeStruct((W2, B), jnp.float32),
            jax.ShapeDtypeStruct((W2, B), jnp.float32),
            jax.ShapeDtypeStruct((W2, B), jnp.float32),
            jax.ShapeDtypeStruct((NSEL, S1 * W2, B), jnp.float32),
            jax.ShapeDtypeStruct((NSEL, S1 * W2, B), jnp.float32),
        ],
    )(timeT, qmat, sraw, praw, oraw, hraw, traw, cee, cer)

    def back(x, m):
        return jnp.transpose(x.reshape(-1, B), (1, 0)).reshape(B, S2, m, D)

    return (back(s_o, 1), back(p_o, 1), back(o_o, 1),
            back(h_o, 4 * NSEL), back(t_o, 4 * NSEL))
